# baseline (speedup 1.0000x reference)
import functools

import jax
import jax.numpy as jnp
from jax import lax
from jax.experimental import pallas as pl
from jax.experimental.pallas import tpu as pltpu
from jax.experimental.pallas import tpu_sc as plsc

BATCH = 16384
EMB = 64
NUM_WORKERS = 32
ROWS_PER_WORKER = BATCH // NUM_WORKERS
IDX_CHUNK = 128
NUM_CHUNKS = ROWS_PER_WORKER // IDX_CHUNK

_sc_mesh = plsc.VectorSubcoreMesh(core_axis_name="c", subcore_axis_name="s")


@functools.partial(
    pl.kernel,
    mesh=_sc_mesh,
    compiler_params=pltpu.CompilerParams(use_tc_tiling_on_sc=False),
    out_type=[
        jax.ShapeDtypeStruct((BATCH, EMB), jnp.float32),
        jax.ShapeDtypeStruct((BATCH, EMB), jnp.float32),
    ],
    scratch_types=[
        pltpu.VMEM((NUM_CHUNKS, IDX_CHUNK), jnp.int32),
        pltpu.VMEM((NUM_CHUNKS, IDX_CHUNK), jnp.int32),
        pltpu.VMEM((ROWS_PER_WORKER, EMB), jnp.float32),
        pltpu.VMEM((ROWS_PER_WORKER, EMB), jnp.float32),
        pltpu.SemaphoreType.DMA,
    ],
)
def _sc_gather(uid_hbm, iid_hbm, utab_hbm, itab_hbm, uout_hbm, iout_hbm,
               uidx_v, iidx_v, urows_v, irows_v, sem):
    wid = lax.axis_index("s") * 2 + lax.axis_index("c")
    base = wid * ROWS_PER_WORKER
    pltpu.sync_copy(uid_hbm.at[wid], uidx_v)
    pltpu.sync_copy(iid_hbm.at[wid], iidx_v)
    copies = []
    for j in range(NUM_CHUNKS):
        dst_u = urows_v.at[pl.ds(j * IDX_CHUNK, IDX_CHUNK)]
        dst_i = irows_v.at[pl.ds(j * IDX_CHUNK, IDX_CHUNK)]
        copies.append(pltpu.async_copy(utab_hbm.at[uidx_v.at[j]], dst_u, sem))
        copies.append(pltpu.async_copy(itab_hbm.at[iidx_v.at[j]], dst_i, sem))
    for c in copies:
        c.wait()
    pltpu.sync_copy(urows_v, uout_hbm.at[pl.ds(base, ROWS_PER_WORKER)])
    pltpu.sync_copy(irows_v, iout_hbm.at[pl.ds(base, ROWS_PER_WORKER)])


def _mlp_body(u_ref, i_ref, w1_ref, b1_ref, w2_ref, b2_ref, o_ref):
    prod = u_ref[...] * i_ref[...]
    h = jnp.dot(prod, w1_ref[...], preferred_element_type=jnp.float32)
    h = jnp.maximum(h + b1_ref[...], 0.0)
    o = jnp.sum(h * w2_ref[...], axis=1) + b2_ref[0, 0]
    o_ref[...] = jax.nn.sigmoid(o)


def kernel(user_ids, item_ids, user_table, item_table, W1, b1, W2, b2):
    uid = user_ids.astype(jnp.int32).reshape(NUM_WORKERS, NUM_CHUNKS, IDX_CHUNK)
    iid = item_ids.astype(jnp.int32).reshape(NUM_WORKERS, NUM_CHUNKS, IDX_CHUNK)
    u_emb, i_emb = _sc_gather(uid, iid, user_table, item_table)

    blk = 2048
    out = pl.pallas_call(
        _mlp_body,
        grid=(BATCH // blk,),
        in_specs=[
            pl.BlockSpec((blk, EMB), lambda b: (b, 0)),
            pl.BlockSpec((blk, EMB), lambda b: (b, 0)),
            pl.BlockSpec((EMB, 32), lambda b: (0, 0)),
            pl.BlockSpec((1, 32), lambda b: (0, 0)),
            pl.BlockSpec((1, 32), lambda b: (0, 0)),
            pl.BlockSpec((1, 1), lambda b: (0, 0)),
        ],
        out_specs=pl.BlockSpec((blk,), lambda b: (b,)),
        out_shape=jax.ShapeDtypeStruct((BATCH,), jnp.float32),
    )(u_emb, i_emb, W1, b1.reshape(1, 32), W2.reshape(1, 32),
      b2.reshape(1, 1))
    return out
